# baseline (speedup 1.0000x reference)
import functools

import jax
import jax.numpy as jnp
from jax.experimental import pallas as pl
from jax.experimental.pallas import tpu as pltpu


def _pick_block(n, want):
    if n % want == 0:
        return want
    return n


def _bcast_spec(shape):
    return pl.BlockSpec(shape, lambda i: (0,) * len(shape))


def _mlp2_kernel(x_ref, w1_ref, b1_ref, w2_ref, b2_ref, o_ref):
    mid = jnp.maximum(
        jnp.dot(x_ref[...], w1_ref[...], preferred_element_type=jnp.float32)
        + b1_ref[...],
        0.0,
    )
    o_ref[...] = (
        jnp.dot(mid, w2_ref[...], preferred_element_type=jnp.float32) + b2_ref[...]
    )


def _mlp2(x, p, block_rows):
    n, din = x.shape
    dmid = p["W1"].shape[1]
    dout = p["W2"].shape[1]
    bn = _pick_block(n, block_rows)
    return pl.pallas_call(
        _mlp2_kernel,
        grid=(n // bn,),
        in_specs=[
            pl.BlockSpec((bn, din), lambda i: (i, 0)),
            _bcast_spec((din, dmid)),
            _bcast_spec((1, dmid)),
            _bcast_spec((dmid, dout)),
            _bcast_spec((1, dout)),
        ],
        out_specs=pl.BlockSpec((bn, dout), lambda i: (i, 0)),
        out_shape=jax.ShapeDtypeStruct((n, dout), jnp.float32),
    )(x, p["W1"], p["b1"].reshape(1, -1), p["W2"], p["b2"].reshape(1, -1))


def _edge_mlp_kernel(
    hs_ref, hd_ref, e_ref, w1s_ref, w1d_ref, w1e_ref, b1_ref, w2_ref, b2_ref, o_ref
):
    pre = (
        jnp.dot(hs_ref[...], w1s_ref[...], preferred_element_type=jnp.float32)
        + jnp.dot(hd_ref[...], w1d_ref[...], preferred_element_type=jnp.float32)
        + jnp.dot(e_ref[...], w1e_ref[...], preferred_element_type=jnp.float32)
        + b1_ref[...]
    )
    mid = jnp.maximum(pre, 0.0)
    o_ref[...] = (
        jnp.dot(mid, w2_ref[...], preferred_element_type=jnp.float32) + b2_ref[...]
    )


def _edge_mlp(hs, hd, e, lp):
    ecount, h = hs.shape
    lp = lp["edge_mlp"]
    w1 = lp["W1"]
    dmid = w1.shape[1]
    dout = lp["W2"].shape[1]
    w1s, w1d, w1e = w1[:h], w1[h : 2 * h], w1[2 * h :]
    be = _pick_block(ecount, 3200)
    return pl.pallas_call(
        _edge_mlp_kernel,
        grid=(ecount // be,),
        in_specs=[
            pl.BlockSpec((be, h), lambda i: (i, 0)),
            pl.BlockSpec((be, h), lambda i: (i, 0)),
            pl.BlockSpec((be, h), lambda i: (i, 0)),
            _bcast_spec((h, dmid)),
            _bcast_spec((h, dmid)),
            _bcast_spec((h, dmid)),
            _bcast_spec((1, dmid)),
            _bcast_spec((dmid, dout)),
            _bcast_spec((1, dout)),
        ],
        out_specs=pl.BlockSpec((be, dout), lambda i: (i, 0)),
        out_shape=jax.ShapeDtypeStruct((ecount, dout), jnp.float32),
    )(
        hs, hd, e, w1s, w1d, w1e,
        lp["b1"].reshape(1, -1), lp["W2"], lp["b2"].reshape(1, -1),
    )


def _node_update_kernel(
    h_ref, agg_ref, w1h_ref, w1a_ref, b1_ref, w2_ref, b2_ref, g_ref, lb_ref, o_ref
):
    h = h_ref[...]
    pre = (
        jnp.dot(h, w1h_ref[...], preferred_element_type=jnp.float32)
        + jnp.dot(agg_ref[...], w1a_ref[...], preferred_element_type=jnp.float32)
        + b1_ref[...]
    )
    mid = jnp.maximum(pre, 0.0)
    upd = jnp.dot(mid, w2_ref[...], preferred_element_type=jnp.float32) + b2_ref[...]
    y = h + upd
    mu = jnp.mean(y, axis=-1, keepdims=True)
    var = jnp.mean((y - mu) ** 2, axis=-1, keepdims=True)
    o_ref[...] = (y - mu) * jax.lax.rsqrt(var + 1e-5) * g_ref[...] + lb_ref[...]


def _node_update(h, agg, lp):
    n, hd = h.shape
    mp = lp["node_mlp"]
    w1 = mp["W1"]
    dmid = w1.shape[1]
    dout = mp["W2"].shape[1]
    w1h, w1a = w1[:hd], w1[hd:]
    bn = _pick_block(n, 2000)
    return pl.pallas_call(
        _node_update_kernel,
        grid=(n // bn,),
        in_specs=[
            pl.BlockSpec((bn, hd), lambda i: (i, 0)),
            pl.BlockSpec((bn, hd), lambda i: (i, 0)),
            _bcast_spec((hd, dmid)),
            _bcast_spec((hd, dmid)),
            _bcast_spec((1, dmid)),
            _bcast_spec((dmid, dout)),
            _bcast_spec((1, dout)),
            _bcast_spec((1, dout)),
            _bcast_spec((1, dout)),
        ],
        out_specs=pl.BlockSpec((bn, dout), lambda i: (i, 0)),
        out_shape=jax.ShapeDtypeStruct((n, dout), jnp.float32),
    )(
        h, agg, w1h, w1a,
        mp["b1"].reshape(1, -1), mp["W2"], mp["b2"].reshape(1, -1),
        lp["ln_g"].reshape(1, -1), lp["ln_b"].reshape(1, -1),
    )


def _head_kernel(h_ref, w1_ref, b1_ref, w2_ref, o_ref):
    mid = jnp.maximum(
        jnp.dot(h_ref[...], w1_ref[...], preferred_element_type=jnp.float32)
        + b1_ref[...],
        0.0,
    )
    o_ref[...] = jnp.dot(mid, w2_ref[...], preferred_element_type=jnp.float32)


def _head(h, p):
    n, hd = h.shape
    dmid = p["W1"].shape[1]
    w2p = jnp.pad(p["W2"], ((0, 0), (0, 128 - p["W2"].shape[1])))
    bn = _pick_block(n, 2000)
    out = pl.pallas_call(
        _head_kernel,
        grid=(n // bn,),
        in_specs=[
            pl.BlockSpec((bn, hd), lambda i: (i, 0)),
            _bcast_spec((hd, dmid)),
            _bcast_spec((1, dmid)),
            _bcast_spec((dmid, 128)),
        ],
        out_specs=pl.BlockSpec((bn, 128), lambda i: (i, 0)),
        out_shape=jax.ShapeDtypeStruct((n, 128), jnp.float32),
    )(h, p["W1"], p["b1"].reshape(1, -1), w2p)
    return out[:, 0] + p["b2"][0]


def kernel(x, edge_index, edge_attr, params):
    h = _mlp2(x, params["node_enc"], 2000)
    e = _mlp2(edge_attr, params["edge_enc"], 3200)
    src = edge_index[0].astype(jnp.int32)
    dst = edge_index[1].astype(jnp.int32)
    n = h.shape[0]
    for lp in params["layers"]:
        hs = jnp.take(h, src, axis=0)
        hd = jnp.take(h, dst, axis=0)
        m = _edge_mlp(hs, hd, e, lp)
        agg = jax.ops.segment_sum(m, dst, num_segments=n)
        h = _node_update(h, agg, lp)
    return _head(h, params["reg_head"])


# speedup vs baseline: 2.6991x; 2.6991x over previous
import functools

import jax
import jax.numpy as jnp
from jax import lax
from jax.experimental import pallas as pl
from jax.experimental.pallas import tpu as pltpu
from jax.experimental.pallas import tpu_sc as plsc

_NC = 2
_NS = 16
_NW = _NC * _NS


def _sc_gather(table, idx2d, nrows):
    v, d = table.shape
    K = 80
    G = 400
    per_w = nrows // _NW
    ngrp = per_w // G
    npair = ngrp // 2
    kj = G // K

    mesh = plsc.VectorSubcoreMesh(core_axis_name="c", subcore_axis_name="s")

    @functools.partial(
        pl.kernel,
        mesh=mesh,
        out_type=jax.ShapeDtypeStruct((nrows, d), table.dtype),
        scratch_types=[
            pltpu.VMEM((16, K), jnp.int32),
            pltpu.VMEM((2, G, d), table.dtype),
            pltpu.SemaphoreType.DMA,
            pltpu.SemaphoreType.DMA,
            pltpu.SemaphoreType.DMA,
            pltpu.SemaphoreType.DMA,
            pltpu.SemaphoreType.DMA,
            pltpu.SemaphoreType.DMA,
        ],
    )
    def k(table_hbm, idx_hbm, out_hbm, idx_v, rows_v, si0, si1, sg0, sg1, so0, so1):
        wid = lax.axis_index("s") * _NC + lax.axis_index("c")
        gbase = wid * ngrp
        si = (si0, si1)
        sg = (sg0, sg1)
        so = (so0, so1)

        def idx_start(g, slot):
            pltpu.async_copy(
                idx_hbm.at[g], idx_v.at[pl.ds(slot * 8, kj)], si[slot]
            )

        def idx_wait(slot):
            pltpu.make_async_copy(
                idx_hbm.at[0], idx_v.at[pl.ds(slot * 8, kj)], si[slot]
            ).wait()

        def gath_start(slot):
            for j in range(kj):
                pltpu.async_copy(
                    table_hbm.at[idx_v.at[slot * 8 + j]],
                    rows_v.at[slot, pl.ds(j * K, K)],
                    sg[slot],
                )

        def gath_wait(slot):
            for j in range(kj):
                pltpu.make_async_copy(
                    table_hbm.at[idx_v.at[slot * 8 + j]],
                    rows_v.at[slot, pl.ds(j * K, K)],
                    sg[slot],
                ).wait()

        def out_start(g, slot):
            pltpu.async_copy(rows_v.at[slot], out_hbm.at[pl.ds(g * G, G)], so[slot])

        def out_wait(slot):
            pltpu.make_async_copy(
                rows_v.at[slot], out_hbm.at[pl.ds(0, G)], so[slot]
            ).wait()

        def pair(p, first, last):
            g0 = gbase + 2 * p
            g1 = g0 + 1
            if not first:
                out_wait(0)
            idx_wait(0)
            gath_start(0)
            if not first:
                out_wait(1)
            idx_wait(1)
            gath_start(1)
            gath_wait(0)
            out_start(g0, 0)
            if not last:
                idx_start(g0 + 2, 0)
            gath_wait(1)
            out_start(g1, 1)
            if not last:
                idx_start(g1 + 2, 1)

        idx_start(gbase, 0)
        idx_start(gbase + 1, 1)
        pair(0, True, npair == 1)

        if npair > 2:
            def body(p, _):
                pair(p, False, False)
                return 0

            lax.fori_loop(1, npair - 1, body, 0)
        if npair > 1:
            pair(npair - 1, False, True)
        out_wait(0)
        out_wait(1)

    return k(table, idx2d)


_NHALF = 5120


def _sc_scatter_add(m, dstidx3d):
    e, d = m.shape
    K = 80
    G = 160
    per_w = e // _NS
    ngrp = per_w // G
    npair = ngrp // 2
    kj = G // K
    rows_sub = _NHALF // _NS
    zrows = 32

    mesh = plsc.VectorSubcoreMesh(core_axis_name="c", subcore_axis_name="s")

    @functools.partial(
        pl.kernel,
        mesh=mesh,
        out_type=jax.ShapeDtypeStruct((2, _NHALF, d), jnp.float32),
        scratch_types=[
            pltpu.VMEM((16, K), jnp.int32),
            pltpu.VMEM((2, G, d), jnp.float32),
            pltpu.VMEM((zrows, d), jnp.float32),
            pltpu.VMEM_SHARED((_NHALF + 8, d), jnp.float32),
            pltpu.SemaphoreType.DMA,
            pltpu.SemaphoreType.DMA,
            pltpu.SemaphoreType.DMA,
            pltpu.SemaphoreType.DMA,
        ],
    )
    def k(m_hbm, idx_hbm, out_hbm, idx_v, m_v, zbuf, acc, si0, si1, sm0, sm1):
        core = lax.axis_index("c")
        sub = lax.axis_index("s")
        gbase = sub * ngrp
        lo = core * _NHALF
        si = (si0, si1)
        sm = (sm0, sm1)

        def in_start(g, slot):
            pltpu.async_copy(
                idx_hbm.at[g], idx_v.at[pl.ds(slot * 8, kj)], si[slot]
            )
            pltpu.async_copy(m_hbm.at[pl.ds(g * G, G)], m_v.at[slot], sm[slot])

        def in_wait(slot):
            pltpu.make_async_copy(
                idx_hbm.at[0], idx_v.at[pl.ds(slot * 8, kj)], si[slot]
            ).wait()
            pltpu.make_async_copy(
                m_hbm.at[pl.ds(0, G)], m_v.at[slot], sm[slot]
            ).wait()

        def localize(slot):
            for j in range(kj):
                for c in range(K // 16):
                    v = idx_v[slot * 8 + j, pl.ds(c * 16, 16)] - lo
                    oob = (v < 0) | (v >= _NHALF)
                    idx_v[slot * 8 + j, pl.ds(c * 16, 16)] = jnp.where(
                        oob, _NHALF, v
                    )

        def scat(slot):
            for j in range(kj):
                pltpu.sync_copy(
                    m_v.at[slot, pl.ds(j * K, K)],
                    acc.at[idx_v.at[slot * 8 + j]],
                    add=True,
                )

        in_start(gbase, 0)
        in_start(gbase + 1, 1)
        for r in range(zrows):
            for j in range(d // 16):
                zbuf[r, pl.ds(j * 16, 16)] = jnp.zeros((16,), jnp.float32)
        for i in range(rows_sub // zrows):
            pltpu.sync_copy(
                zbuf, acc.at[pl.ds(sub * rows_sub + i * zrows, zrows)]
            )
        plsc.subcore_barrier()

        def pair(p, last):
            g0 = gbase + 2 * p
            g1 = g0 + 1
            in_wait(0)
            localize(0)
            scat(0)
            if not last:
                in_start(g0 + 2, 0)
            in_wait(1)
            localize(1)
            scat(1)
            if not last:
                in_start(g1 + 2, 1)

        pair(0, npair == 1)
        if npair > 2:
            def body(p, _):
                pair(p, False)
                return 0

            lax.fori_loop(1, npair - 1, body, 0)
        if npair > 1:
            pair(npair - 1, True)

        plsc.subcore_barrier()
        hrow = rows_sub // 2
        for i in range(2):
            r0 = sub * rows_sub + i * hrow
            pltpu.sync_copy(acc.at[pl.ds(r0, hrow)], m_v.at[0, pl.ds(0, hrow)])
            pltpu.sync_copy(
                m_v.at[0, pl.ds(0, hrow)], out_hbm.at[core, pl.ds(r0, hrow)]
            )

    return k(m, dstidx3d)


def _pick_block(n, want):
    if n % want == 0:
        return want
    return n


def _bcast_spec(shape):
    return pl.BlockSpec(shape, lambda i: (0,) * len(shape))


_BF = jnp.bfloat16


def _dot(a, b):
    return jnp.dot(a.astype(_BF), b, preferred_element_type=jnp.float32)


def _mlp2_kernel(x_ref, w1_ref, b1_ref, w2_ref, b2_ref, o_ref):
    mid = jnp.maximum(_dot(x_ref[...], w1_ref[...]) + b1_ref[...], 0.0)
    o_ref[...] = _dot(mid, w2_ref[...]) + b2_ref[...]


def _mlp2(x, p, block_rows):
    n, din = x.shape
    dmid = p["W1"].shape[1]
    dout = p["W2"].shape[1]
    bn = _pick_block(n, block_rows)
    return pl.pallas_call(
        _mlp2_kernel,
        grid=(n // bn,),
        in_specs=[
            pl.BlockSpec((bn, din), lambda i: (i, 0)),
            _bcast_spec((din, dmid)),
            _bcast_spec((1, dmid)),
            _bcast_spec((dmid, dout)),
            _bcast_spec((1, dout)),
        ],
        out_specs=pl.BlockSpec((bn, dout), lambda i: (i, 0)),
        out_shape=jax.ShapeDtypeStruct((n, dout), jnp.float32),
    )(x, p["W1"].astype(_BF), p["b1"].reshape(1, -1),
      p["W2"].astype(_BF), p["b2"].reshape(1, -1))


def _edge_mlp_kernel(
    hs_ref, hd_ref, e_ref, w1_ref, b1_ref, w2_ref, b2_ref, o_ref
):
    xcat = jnp.concatenate(
        [hs_ref[...].astype(_BF), hd_ref[...].astype(_BF)], axis=-1
    )
    h = hs_ref.shape[1]
    pre = _dot(xcat, w1_ref[: 2 * h]) + _dot(e_ref[...], w1_ref[2 * h :])
    mid = jnp.maximum(pre + b1_ref[...], 0.0)
    o_ref[...] = _dot(mid, w2_ref[...]) + b2_ref[...]


def _edge_mlp(ghs, e, lp):
    ecount, h = e.shape
    lp = lp["edge_mlp"]
    w1 = lp["W1"]
    dmid = w1.shape[1]
    dout = lp["W2"].shape[1]
    be = _pick_block(ecount, 3200)
    nblk = ecount // be
    return pl.pallas_call(
        _edge_mlp_kernel,
        grid=(nblk,),
        in_specs=[
            pl.BlockSpec((be, h), lambda i: (i, 0)),
            pl.BlockSpec((be, h), lambda i: (i + nblk, 0)),
            pl.BlockSpec((be, h), lambda i: (i, 0)),
            _bcast_spec((3 * h, dmid)),
            _bcast_spec((1, dmid)),
            _bcast_spec((dmid, dout)),
            _bcast_spec((1, dout)),
        ],
        out_specs=pl.BlockSpec((be, dout), lambda i: (i, 0)),
        out_shape=jax.ShapeDtypeStruct((ecount, dout), jnp.float32),
    )(
        ghs, ghs, e, w1.astype(_BF),
        lp["b1"].reshape(1, -1), lp["W2"].astype(_BF), lp["b2"].reshape(1, -1),
    )


def _fold_mean(y):
    s = y
    w = y.shape[-1]
    while w > 1:
        w //= 2
        s = s[:, :w] + s[:, w:]
    return s * (1.0 / y.shape[-1])


def _node_update_kernel(
    h_ref, agg_ref, w1_ref, b1_ref, w2_ref, b2_ref, o_ref
):
    h = h_ref[...]
    xcat = jnp.concatenate([h.astype(_BF), agg_ref[...].astype(_BF)], axis=-1)
    mid = jnp.maximum(_dot(xcat, w1_ref[...]) + b1_ref[...], 0.0)
    upd = _dot(mid, w2_ref[...]) + b2_ref[...]
    o_ref[...] = h + upd


def _node_update(h, agg, lp):
    n, hd = h.shape
    mp = lp["node_mlp"]
    w1 = mp["W1"]
    dmid = w1.shape[1]
    dout = mp["W2"].shape[1]
    bn = _pick_block(n, 2000)
    out = pl.pallas_call(
        _node_update_kernel,
        grid=(n // bn,),
        in_specs=[
            pl.BlockSpec((bn, hd), lambda i: (i, 0)),
            pl.BlockSpec((bn, hd), lambda i: (i, 0)),
            _bcast_spec((2 * hd, dmid)),
            _bcast_spec((1, dmid)),
            _bcast_spec((dmid, dout)),
            _bcast_spec((1, dout)),
        ],
        out_specs=pl.BlockSpec((bn, dout), lambda i: (i, 0)),
        out_shape=jax.ShapeDtypeStruct((n, dout), jnp.float32),
    )(
        h, agg, w1.astype(_BF),
        mp["b1"].reshape(1, -1), mp["W2"].astype(_BF), mp["b2"].reshape(1, -1),
    )
    y = out
    mu = y.mean(-1, keepdims=True)
    var = y.var(-1, keepdims=True)
    return (y - mu) / jnp.sqrt(var + 1e-5) * lp["ln_g"] + lp["ln_b"]


def _head_kernel(h_ref, w1_ref, b1_ref, w2_ref, o_ref):
    mid = jnp.maximum(_dot(h_ref[...], w1_ref[...]) + b1_ref[...], 0.0)
    o_ref[...] = _dot(mid, w2_ref[...])


def _head(h, p):
    n, hd = h.shape
    dmid = p["W1"].shape[1]
    w2p = jnp.pad(p["W2"], ((0, 0), (0, 128 - p["W2"].shape[1])))
    bn = _pick_block(n, 2000)
    out = pl.pallas_call(
        _head_kernel,
        grid=(n // bn,),
        in_specs=[
            pl.BlockSpec((bn, hd), lambda i: (i, 0)),
            _bcast_spec((hd, dmid)),
            _bcast_spec((1, dmid)),
            _bcast_spec((dmid, 128)),
        ],
        out_specs=pl.BlockSpec((bn, 128), lambda i: (i, 0)),
        out_shape=jax.ShapeDtypeStruct((n, 128), jnp.float32),
    )(h, p["W1"].astype(_BF), p["b1"].reshape(1, -1), w2p.astype(_BF))
    return out[:, 0] + p["b2"][0]


_NPAD = 10240


def kernel(x, edge_index, edge_attr, params):
    h = _mlp2(x, params["node_enc"], 2000)
    e = _mlp2(edge_attr, params["edge_enc"], 3200)
    ei = edge_index.astype(jnp.int32)
    ecount = ei.shape[1]
    idx3d = ei.reshape(2 * ecount // 400, 5, 80)
    dst3d = ei[1].reshape(ecount // 160, 2, 80)
    e = e.astype(_BF)
    for lp in params["layers"]:
        ghs = _sc_gather(h, idx3d, 2 * ecount)
        m = _edge_mlp(ghs, e, lp)
        agg2 = jax.ops.segment_sum(m, ei[1], num_segments=2 * _NHALF)
        h = _node_update(h, agg2.reshape(2 * _NHALF, -1), lp)
    return _head(h, params["reg_head"])
